# baseline (speedup 1.0000x reference)
import functools

import jax
import jax.numpy as jnp
from jax import lax
from jax.experimental import pallas as pl
from jax.experimental.pallas import tpu as pltpu
from jax.experimental.pallas import tpu_sc as plsc

BATCH = 16384
D = 16
NC = 2
NS = 16
NW = NC * NS
PER_W = BATCH // NW
CHUNK = 128
NCH = PER_W // CHUNK
NBLK = PER_W // D


def _sc_body(uid_hbm, iid_hbm, umf_hbm, imf_hbm, umlp_hbm, imlp_hbm,
             w_hbm, c_hbm, out_hbm,
             uidx_v, iidx_v, umf_v, imf_v, umlp_v, imlp_v, w_v, c_v,
             out_v, sem):
    cid = lax.axis_index("c")
    sid = lax.axis_index("s")
    wid = sid * NC + cid

    pltpu.sync_copy(uid_hbm.at[wid], uidx_v)
    pltpu.sync_copy(iid_hbm.at[wid], iidx_v)
    pltpu.sync_copy(w_hbm, w_v)
    pltpu.sync_copy(c_hbm, c_v)

    copies = []
    for j in range(NCH):
        sl = pl.ds(j * CHUNK, CHUNK)
        for tbl, idx, dst in ((umf_hbm, uidx_v, umf_v),
                              (imf_hbm, iidx_v, imf_v),
                              (umlp_hbm, uidx_v, umlp_v),
                              (imlp_hbm, iidx_v, imlp_v)):
            copies.append(pltpu.async_copy(tbl.at[idx.at[j]], dst.at[sl], sem))
    for cp in copies:
        cp.wait()

    iota = lax.iota(jnp.int32, D)
    c_splat = c_v[...]
    wmf = w_v[0]
    wa = w_v[1]
    wb = w_v[2]

    def blk(b, _):
        base = b * D
        acc = c_splat
        for l in range(D):
            e = base + l
            t = umf_v[e] * imf_v[e] * wmf + umlp_v[e] * wa + imlp_v[e] * wb
            acc = jnp.where(iota == l, jnp.sum(t), acc)
        out_v[pl.ds(base, D)] = 1.0 / (1.0 + jnp.exp(-(acc + c_splat)))
        return _

    lax.fori_loop(0, NBLK, blk, 0, unroll=False)
    pltpu.sync_copy(out_v, out_hbm.at[wid])


@functools.partial(
    pl.kernel,
    out_type=jax.ShapeDtypeStruct((NW, PER_W), jnp.float32),
    mesh=plsc.VectorSubcoreMesh(core_axis_name="c", subcore_axis_name="s"),
    compiler_params=pltpu.CompilerParams(
        needs_layout_passes=False, use_tc_tiling_on_sc=False),
    scratch_types=[
        pltpu.VMEM((NCH, CHUNK), jnp.int32),
        pltpu.VMEM((NCH, CHUNK), jnp.int32),
        pltpu.VMEM((PER_W, D), jnp.float32),
        pltpu.VMEM((PER_W, D), jnp.float32),
        pltpu.VMEM((PER_W, D), jnp.float32),
        pltpu.VMEM((PER_W, D), jnp.float32),
        pltpu.VMEM((3, D), jnp.float32),
        pltpu.VMEM((D,), jnp.float32),
        pltpu.VMEM((PER_W,), jnp.float32),
        pltpu.SemaphoreType.DMA,
    ],
)
def _neumf_sc(uid_hbm, iid_hbm, umf_hbm, imf_hbm, umlp_hbm, imlp_hbm,
              w_hbm, c_hbm, out_hbm, *scratch):
    _sc_body(uid_hbm, iid_hbm, umf_hbm, imf_hbm, umlp_hbm, imlp_hbm,
             w_hbm, c_hbm, out_hbm, *scratch)


def kernel(X, user_mf, item_mf, user_mlp, item_mlp, W_mlp, b_mlp, W_pred, b_pred):
    Xi = X.astype(jnp.int32)
    uid = Xi[:, 0].reshape(NW, NCH, CHUNK)
    iid = Xi[:, 1].reshape(NW, NCH, CHUNK)

    h = W_pred[D:, 0]
    a = W_mlp[:D, :] @ h
    b = W_mlp[D:, :] @ h
    c = b_mlp @ h + b_pred[0]
    wmf = W_pred[:D, 0]
    w_vecs = jnp.stack([wmf, a, b]).astype(jnp.float32)
    c_vec = jnp.full((D,), c, jnp.float32)

    out = _neumf_sc(uid, iid, user_mf, item_mf, user_mlp, item_mlp,
                    w_vecs, c_vec)
    return out.reshape(BATCH, 1)
